# baseline (speedup 1.0000x reference)
import functools

import jax
import jax.numpy as jnp
from jax import lax
from jax.experimental import pallas as pl
from jax.experimental.pallas import tpu as pltpu
from jax.experimental.pallas import tpu_sc as plsc

N_ROWS = 320000
D = 128
N_SEG = 1024
NC = 2
NS = 16
NW = NC * NS
ROWS_PER_W = N_ROWS // NW
CHUNK = 80
NCHUNK = ROWS_PER_W // CHUNK
NBUF = 10
SEG_PER_TILE = N_SEG // NS
NLANE = 16
NV = D // NLANE
UNROLL = 8
TEC_B = tuple(range(NBUF))


def _sc_partials(X, I32, Z):
    mesh = plsc.VectorSubcoreMesh(core_axis_name="c", subcore_axis_name="s")

    @functools.partial(
        pl.kernel,
        mesh=mesh,
        out_type=jax.ShapeDtypeStruct((NC, N_SEG, D), jnp.float32),
        scratch_types=[
            pltpu.VMEM((NCHUNK, CHUNK), jnp.int32),
            pltpu.VMEM((NBUF, CHUNK, D), jnp.float32),
            pltpu.VMEM((NLANE, D), jnp.float32),
            pltpu.VMEM((NLANE,), jnp.int32),
            pltpu.VMEM_SHARED((N_SEG, D), jnp.float32),
        ] + [pltpu.SemaphoreType.DMA] * (2 * NBUF),
    )
    def k(x_hbm, i_hbm, z_hbm, out_hbm, idx_v, data_v, stage_v, sidx_v,
          acc_sh, *sems):
        c = lax.axis_index("c")
        s = lax.axis_index("s")
        wid = c * NS + s
        base = wid * ROWS_PER_W
        zf = jnp.zeros((NLANE,), jnp.float32)
        zi = jnp.zeros((NLANE,), jnp.int32)
        lanes = lax.iota(jnp.int32, NLANE)

        def clear_stage():
            for r in range(NLANE):
                for v in range(NV):
                    stage_v[r, pl.ds(NLANE * v, NLANE)] = zf
            sidx_v[...] = zi

        pltpu.sync_copy(z_hbm.at[pl.ds(s * SEG_PER_TILE, SEG_PER_TILE)],
                        acc_sh.at[pl.ds(s * SEG_PER_TILE, SEG_PER_TILE)])
        pltpu.sync_copy(i_hbm.at[wid], idx_v)
        clear_stage()
        plsc.subcore_barrier()

        def fill(j, b):
            pltpu.async_copy(x_hbm.at[pl.ds(base + j * CHUNK, CHUNK)],
                             data_v.at[b], sems[b])

        def wait_fill(b):
            pltpu.make_async_copy(x_hbm.at[pl.ds(0, CHUNK)], data_v.at[b],
                                  sems[b]).wait()

        def scatter(j, b):
            pltpu.async_copy(data_v.at[b], acc_sh.at[idx_v.at[j]],
                             sems[NBUF + b], add=True)

        def wait_scatter(b):
            pltpu.make_async_copy(x_hbm.at[pl.ds(0, CHUNK)], data_v.at[b],
                                  sems[NBUF + b]).wait()

        def first_last(j):
            return (idx_v[j, pl.ds(0, NLANE)][0],
                    idx_v[j, pl.ds(CHUNK - NLANE, NLANE)][NLANE - 1])

        def flush():
            pltpu.sync_copy(stage_v, acc_sh.at[sidx_v], add=True)
            clear_stage()

        def append1(lane, rows, seg):
            lane = lax.cond(lane > NLANE - 1,
                            lambda: (flush(), jnp.int32(0))[1],
                            lambda: lane)
            for v in range(NV):
                stage_v[lane, pl.ds(NLANE * v, NLANE)] = rows[v]
            sidx_v[...] = jnp.where(lanes == lane, seg, sidx_v[...])
            return lane + 1

        def reduce_chunk(j, b, lane, seg):
            def rbody(r8, acc):
                for u in range(UNROLL):
                    r = r8 * UNROLL + u
                    acc = tuple(
                        acc[v] + data_v[b, r, pl.ds(NLANE * v, NLANE)]
                        for v in range(NV))
                return acc
            tot = lax.fori_loop(0, CHUNK // UNROLL, rbody, (zf,) * NV)
            return append1(lane, tot, seg)

        def retire(j, b):
            if b in TEC_B:
                sm, sx = first_last(j)

                @pl.when(sm != sx)
                def _():
                    wait_scatter(b)
            else:
                wait_scatter(b)

        def step(j, b, bp):
            @pl.when(j > 0)
            def _():
                retire(j - 1, bp)

                @pl.when(j - 1 + NBUF < NCHUNK)
                def _():
                    fill(j - 1 + NBUF, bp)

            wait_fill(b)
            return j, b

        def full_step(j, b, lane):
            step(j, b, (b - 1) % NBUF)
            sm, sx = first_last(j)

            def tec_path(lane, j=j, b=b, sm=sm):
                return reduce_chunk(j, b, lane, sm)

            def dma_path(lane, j=j, b=b):
                scatter(j, b)
                return lane

            return lax.cond(sm == sx, tec_path, dma_path, lane)

        def body(j0, lane):
            for b in range(NBUF):
                lane = full_step(j0 + b, b, lane)
            return lane

        NFULL = (NCHUNK // NBUF) * NBUF
        for b in range(NBUF):
            fill(b, b)
        lane = lax.fori_loop(0, NFULL // NBUF,
                             lambda i, ln: body(i * NBUF, ln), jnp.int32(0))
        for j in range(NFULL, NCHUNK):
            lane = full_step(jnp.int32(j), j % NBUF, lane)
        retire(jnp.int32(NCHUNK - 1), (NCHUNK - 1) % NBUF)
        flush()
        plsc.subcore_barrier()
        pltpu.sync_copy(acc_sh.at[pl.ds(s * SEG_PER_TILE, SEG_PER_TILE)],
                        out_hbm.at[c, pl.ds(s * SEG_PER_TILE, SEG_PER_TILE)])

    return k(X, I32.reshape(NW, NCHUNK, CHUNK), Z)


def _combine(partials):
    def body(p_ref, o_ref):
        o_ref[...] = p_ref[0] + p_ref[1]

    return pl.pallas_call(
        body,
        out_shape=jax.ShapeDtypeStruct((N_SEG, D), jnp.float32),
    )(partials)


def kernel(X, I):
    if I.ndim == 2:
        I = I[:, 0]
    I32 = I.astype(jnp.int32)
    Z = jnp.zeros((N_SEG, D), jnp.float32)
    partials = _sc_partials(X, I32, Z)
    return _combine(partials)


# speedup vs baseline: 1.3620x; 1.3620x over previous
import functools

import jax
import jax.numpy as jnp
from jax import lax
from jax.experimental import pallas as pl
from jax.experimental.pallas import tpu as pltpu
from jax.experimental.pallas import tpu_sc as plsc

N_ROWS = 320000
D = 128
N_SEG = 1024
NC = 2
NS = 16
NW = NC * NS
ROWS_PER_W = N_ROWS // NW
CHUNK = 80
NCHUNK = ROWS_PER_W // CHUNK
NBUF = 5
SEG_PER_TILE = N_SEG // NS
NLANE = 16
NV = D // NLANE
UNROLL = 8
TEC_B = tuple(range(NBUF))


def _sc_partials(X, I32, Z):
    mesh = plsc.VectorSubcoreMesh(core_axis_name="c", subcore_axis_name="s")

    @functools.partial(
        pl.kernel,
        mesh=mesh,
        out_type=jax.ShapeDtypeStruct((NC, N_SEG, D), jnp.float32),
        scratch_types=[
            pltpu.VMEM((NCHUNK, CHUNK), jnp.int32),
            pltpu.VMEM((NBUF, CHUNK, D), jnp.float32),
            pltpu.VMEM((NLANE, D), jnp.float32),
            pltpu.VMEM((NLANE,), jnp.int32),
            pltpu.VMEM_SHARED((N_SEG, D), jnp.float32),
        ] + [pltpu.SemaphoreType.DMA] * (2 * NBUF),
    )
    def k(x_hbm, i_hbm, z_hbm, out_hbm, idx_v, data_v, stage_v, sidx_v,
          acc_sh, *sems):
        c = lax.axis_index("c")
        s = lax.axis_index("s")
        wid = c * NS + s
        base = wid * ROWS_PER_W
        zf = jnp.zeros((NLANE,), jnp.float32)
        zi = jnp.zeros((NLANE,), jnp.int32)
        lanes = lax.iota(jnp.int32, NLANE)

        def clear_stage():
            for r in range(NLANE):
                for v in range(NV):
                    stage_v[r, pl.ds(NLANE * v, NLANE)] = zf
            sidx_v[...] = zi

        pltpu.sync_copy(z_hbm.at[pl.ds(s * SEG_PER_TILE, SEG_PER_TILE)],
                        acc_sh.at[pl.ds(s * SEG_PER_TILE, SEG_PER_TILE)])
        pltpu.sync_copy(i_hbm.at[wid], idx_v)
        clear_stage()
        plsc.subcore_barrier()

        def fill(j, b):
            pltpu.async_copy(x_hbm.at[pl.ds(base + j * CHUNK, CHUNK)],
                             data_v.at[b], sems[b])

        def wait_fill(b):
            pltpu.make_async_copy(x_hbm.at[pl.ds(0, CHUNK)], data_v.at[b],
                                  sems[b]).wait()

        def scatter(j, b):
            pltpu.async_copy(data_v.at[b], acc_sh.at[idx_v.at[j]],
                             sems[NBUF + b], add=True)

        def wait_scatter(b):
            pltpu.make_async_copy(x_hbm.at[pl.ds(0, CHUNK)], data_v.at[b],
                                  sems[NBUF + b]).wait()

        def first_last(j):
            return (idx_v[j, pl.ds(0, NLANE)][0],
                    idx_v[j, pl.ds(CHUNK - NLANE, NLANE)][NLANE - 1])

        def flush():
            pltpu.sync_copy(stage_v, acc_sh.at[sidx_v], add=True)
            clear_stage()

        def append1(lane, rows, seg):
            lane = lax.cond(lane > NLANE - 1,
                            lambda: (flush(), jnp.int32(0))[1],
                            lambda: lane)
            for v in range(NV):
                stage_v[lane, pl.ds(NLANE * v, NLANE)] = rows[v]
            sidx_v[...] = jnp.where(lanes == lane, seg, sidx_v[...])
            return lane + 1

        def reduce_chunk(j, b, lane, seg):
            def rbody(r8, acc):
                for u in range(UNROLL):
                    r = r8 * UNROLL + u
                    acc = tuple(
                        acc[v] + data_v[b, r, pl.ds(NLANE * v, NLANE)]
                        for v in range(NV))
                return acc
            tot = lax.fori_loop(0, CHUNK // UNROLL, rbody, (zf,) * NV)
            return append1(lane, tot, seg)

        def retire(j, b):
            if b in TEC_B:
                sm, sx = first_last(j)

                @pl.when(sm != sx)
                def _():
                    wait_scatter(b)
            else:
                wait_scatter(b)

        def step(j, b, bp):
            @pl.when(j > 0)
            def _():
                retire(j - 1, bp)

                @pl.when(j - 1 + NBUF < NCHUNK)
                def _():
                    fill(j - 1 + NBUF, bp)

            wait_fill(b)
            return j, b

        def full_step(j, b, lane):
            step(j, b, (b - 1) % NBUF)
            sm, sx = first_last(j)

            def tec_path(lane, j=j, b=b, sm=sm):
                return reduce_chunk(j, b, lane, sm)

            def dma_path(lane, j=j, b=b):
                scatter(j, b)
                return lane

            return lax.cond(sm == sx, tec_path, dma_path, lane)

        def body(j0, lane):
            for b in range(NBUF):
                lane = full_step(j0 + b, b, lane)
            return lane

        NFULL = (NCHUNK // NBUF) * NBUF
        for b in range(NBUF):
            fill(b, b)
        lane = lax.fori_loop(0, NFULL // NBUF,
                             lambda i, ln: body(i * NBUF, ln), jnp.int32(0))
        for j in range(NFULL, NCHUNK):
            lane = full_step(jnp.int32(j), j % NBUF, lane)
        retire(jnp.int32(NCHUNK - 1), (NCHUNK - 1) % NBUF)
        flush()
        plsc.subcore_barrier()
        pltpu.sync_copy(acc_sh.at[pl.ds(s * SEG_PER_TILE, SEG_PER_TILE)],
                        out_hbm.at[c, pl.ds(s * SEG_PER_TILE, SEG_PER_TILE)])

    return k(X, I32.reshape(NW, NCHUNK, CHUNK), Z)


def _combine(partials):
    def body(p_ref, o_ref):
        o_ref[...] = p_ref[0] + p_ref[1]

    return pl.pallas_call(
        body,
        out_shape=jax.ShapeDtypeStruct((N_SEG, D), jnp.float32),
    )(partials)


def kernel(X, I):
    if I.ndim == 2:
        I = I[:, 0]
    I32 = I.astype(jnp.int32)
    Z = jnp.zeros((N_SEG, D), jnp.float32)
    partials = _sc_partials(X, I32, Z)
    return _combine(partials)


# speedup vs baseline: 1.3716x; 1.0071x over previous
import functools

import jax
import jax.numpy as jnp
from jax import lax
from jax.experimental import pallas as pl
from jax.experimental.pallas import tpu as pltpu
from jax.experimental.pallas import tpu_sc as plsc

N_ROWS = 320000
D = 128
N_SEG = 1024
NC = 2
NS = 16
NW = NC * NS
ROWS_PER_W = N_ROWS // NW
CHUNK = 80
NCHUNK = ROWS_PER_W // CHUNK
NBUF = 5
SEG_PER_TILE = N_SEG // NS
NLANE = 16
NV = D // NLANE
UNROLL = 8
TEC_B = tuple(range(NBUF))


def _sc_partials(X, I32, Z):
    mesh = plsc.VectorSubcoreMesh(core_axis_name="c", subcore_axis_name="s")

    @functools.partial(
        pl.kernel,
        mesh=mesh,
        compiler_params=pltpu.CompilerParams(use_tc_tiling_on_sc=False),
        out_type=jax.ShapeDtypeStruct((NC, N_SEG, D), jnp.float32),
        scratch_types=[
            pltpu.VMEM((NCHUNK, CHUNK), jnp.int32),
            pltpu.VMEM((NBUF, CHUNK, D), jnp.float32),
            pltpu.VMEM((NLANE, D), jnp.float32),
            pltpu.VMEM((NLANE,), jnp.int32),
            pltpu.VMEM_SHARED((N_SEG, D), jnp.float32),
        ] + [pltpu.SemaphoreType.DMA] * (2 * NBUF),
    )
    def k(x_hbm, i_hbm, z_hbm, out_hbm, idx_v, data_v, stage_v, sidx_v,
          acc_sh, *sems):
        c = lax.axis_index("c")
        s = lax.axis_index("s")
        wid = c * NS + s
        base = wid * ROWS_PER_W
        zf = jnp.zeros((NLANE,), jnp.float32)
        zi = jnp.zeros((NLANE,), jnp.int32)
        lanes = lax.iota(jnp.int32, NLANE)

        def clear_stage():
            for r in range(NLANE):
                for v in range(NV):
                    stage_v[r, pl.ds(NLANE * v, NLANE)] = zf
            sidx_v[...] = zi

        pltpu.sync_copy(z_hbm.at[pl.ds(s * SEG_PER_TILE, SEG_PER_TILE)],
                        acc_sh.at[pl.ds(s * SEG_PER_TILE, SEG_PER_TILE)])
        pltpu.sync_copy(i_hbm.at[wid], idx_v)
        clear_stage()
        plsc.subcore_barrier()

        def fill(j, b):
            pltpu.async_copy(x_hbm.at[pl.ds(base + j * CHUNK, CHUNK)],
                             data_v.at[b], sems[b])

        def wait_fill(b):
            pltpu.make_async_copy(x_hbm.at[pl.ds(0, CHUNK)], data_v.at[b],
                                  sems[b]).wait()

        def scatter(j, b):
            pltpu.async_copy(data_v.at[b], acc_sh.at[idx_v.at[j]],
                             sems[NBUF + b], add=True)

        def wait_scatter(b):
            pltpu.make_async_copy(x_hbm.at[pl.ds(0, CHUNK)], data_v.at[b],
                                  sems[NBUF + b]).wait()

        def first_last(j):
            return (idx_v[j, pl.ds(0, NLANE)][0],
                    idx_v[j, pl.ds(CHUNK - NLANE, NLANE)][NLANE - 1])

        def flush():
            pltpu.sync_copy(stage_v, acc_sh.at[sidx_v], add=True)
            clear_stage()

        def append1(lane, rows, seg):
            lane = lax.cond(lane > NLANE - 1,
                            lambda: (flush(), jnp.int32(0))[1],
                            lambda: lane)
            for v in range(NV):
                stage_v[lane, pl.ds(NLANE * v, NLANE)] = rows[v]
            sidx_v[...] = jnp.where(lanes == lane, seg, sidx_v[...])
            return lane + 1

        def reduce_chunk(j, b, lane, seg):
            def rbody(r8, acc):
                for u in range(UNROLL):
                    r = r8 * UNROLL + u
                    acc = tuple(
                        acc[v] + data_v[b, r, pl.ds(NLANE * v, NLANE)]
                        for v in range(NV))
                return acc
            tot = lax.fori_loop(0, CHUNK // UNROLL, rbody, (zf,) * NV)
            return append1(lane, tot, seg)

        def retire(j, b):
            if b in TEC_B:
                sm, sx = first_last(j)

                @pl.when(sm != sx)
                def _():
                    wait_scatter(b)
            else:
                wait_scatter(b)

        def step(j, b, bp):
            @pl.when(j > 0)
            def _():
                retire(j - 1, bp)

                @pl.when(j - 1 + NBUF < NCHUNK)
                def _():
                    fill(j - 1 + NBUF, bp)

            wait_fill(b)
            return j, b

        def full_step(j, b, lane):
            step(j, b, (b - 1) % NBUF)
            sm, sx = first_last(j)

            def tec_path(lane, j=j, b=b, sm=sm):
                return reduce_chunk(j, b, lane, sm)

            def dma_path(lane, j=j, b=b):
                scatter(j, b)
                return lane

            return lax.cond(sm == sx, tec_path, dma_path, lane)

        def body(j0, lane):
            for b in range(NBUF):
                lane = full_step(j0 + b, b, lane)
            return lane

        NFULL = (NCHUNK // NBUF) * NBUF
        for b in range(NBUF):
            fill(b, b)
        lane = lax.fori_loop(0, NFULL // NBUF,
                             lambda i, ln: body(i * NBUF, ln), jnp.int32(0))
        for j in range(NFULL, NCHUNK):
            lane = full_step(jnp.int32(j), j % NBUF, lane)
        retire(jnp.int32(NCHUNK - 1), (NCHUNK - 1) % NBUF)
        flush()
        plsc.subcore_barrier()
        pltpu.sync_copy(acc_sh.at[pl.ds(s * SEG_PER_TILE, SEG_PER_TILE)],
                        out_hbm.at[c, pl.ds(s * SEG_PER_TILE, SEG_PER_TILE)])

    return k(X, I32.reshape(NW, NCHUNK, CHUNK), Z)


def _combine(partials):
    def body(p_ref, o_ref):
        o_ref[...] = p_ref[0] + p_ref[1]

    return pl.pallas_call(
        body,
        out_shape=jax.ShapeDtypeStruct((N_SEG, D), jnp.float32),
    )(partials)


def kernel(X, I):
    if I.ndim == 2:
        I = I[:, 0]
    I32 = I.astype(jnp.int32)
    Z = jnp.zeros((N_SEG, D), jnp.float32)
    partials = _sc_partials(X, I32, Z)
    return _combine(partials)
